# baseline (device time: 25731 ns/iter reference)
import os

import jax
import jax.numpy as jnp
from jax import lax
from jax.experimental import pallas as pl
from jax.experimental.pallas import tpu as pltpu

_SKIP_COMM = os.environ.get("SKIP_COMM") == "1"

N_DEV = 16
STAGE_MASKS = (8, 3, 4, 1)
N_STAGES = len(STAGE_MASKS)

B, Sq, Hq, Hkv, Dh = 2, 128, 8, 2, 64
D = Hq * Dh
GROUP = Hq // Hkv
GSQ = GROUP * Sq
PR = Dh + 2
SCALE = 0.125
STREAMS = tuple((b, g) for b in range(B) for g in range(Hkv))


def kernel(x, Wq, Wo, K_ext, V_ext):
    def body(x_ref, wq_ref, wo_ref, k_ref, v_ref, out_ref,
             acc_ref, stats_ref, rbuf_ref,
             kbuf_ref, vbuf_ref, qg_ref, obuf_ref,
             send_o, recv_o):
        my = lax.axis_index("i")

        if not _SKIP_COMM:
            barrier_sem = pltpu.get_barrier_semaphore()
            for mask in STAGE_MASKS:
                pl.semaphore_signal(
                    barrier_sem, inc=1,
                    device_id=(my ^ mask,),
                    device_id_type=pl.DeviceIdType.MESH,
                )

        def partial(b):
            for g in range(Hkv):
                kbuf_ref[b, g] = k_ref[b, :, g, :]
                vbuf_ref[b, g] = v_ref[b, :, g, :]
            q_all = lax.dot_general(
                x_ref[b], wq_ref[...], (((1,), (0,)), ((), ())),
            ) * SCALE
            for h in range(Hq):
                g, hh = divmod(h, GROUP)
                qg_ref[b, g, hh * Sq:(hh + 1) * Sq, :] = (
                    q_all[:, h * Dh:(h + 1) * Dh]
                )
            st = lax.dot_general(
                kbuf_ref[b], qg_ref[b], (((2,), (2,)), ((0,), (0,))),
            )
            m16 = jnp.max(st, axis=1, keepdims=True).astype(jnp.bfloat16)
            m = m16.astype(jnp.float32)
            p = jnp.exp(st - m)
            l = jnp.sum(p, axis=1, keepdims=True)
            ot = lax.dot_general(
                vbuf_ref[b], p, (((1,), (1,)), ((0,), (0,))),
            )
            acc_ref[0, b, :, 0:Dh, :] = ot.astype(jnp.bfloat16)
            acc_ref[0, b, :, Dh:Dh + 1, :] = m16
            acc_ref[0, b, :, Dh + 1:Dh + 2, :] = l.astype(jnp.bfloat16)
            stats_ref[b, :, 0] = m
            stats_ref[b, :, 1] = l

        def send(s, b, g):
            partner = my ^ STAGE_MASKS[s]
            out_rdma = pltpu.make_async_remote_copy(
                src_ref=acc_ref.at[s % 2, b, g],
                dst_ref=rbuf_ref.at[s * B + b, g],
                send_sem=send_o.at[s, b, g],
                recv_sem=recv_o.at[s, b, g],
                device_id=(partner,),
                device_id_type=pl.DeviceIdType.MESH,
            )
            out_rdma.start()
            return out_rdma

        def combine(s, b, g, out_rdma, prev):
            p = s % 2
            if prev is not None:
                prev.wait_send()
            out_rdma.wait_recv()
            m_a = stats_ref[b, g, 0]
            l_a = stats_ref[b, g, 1]
            m_b = rbuf_ref[s * B + b, g, Dh:Dh + 1, :].astype(jnp.float32)
            l_b = rbuf_ref[s * B + b, g, Dh + 1:Dh + 2, :].astype(jnp.float32)
            m_n = jnp.maximum(m_a, m_b)
            a_a = jnp.exp(m_a - m_n)
            a_b = jnp.exp(m_b - m_n)
            l_n = l_a * a_a + l_b * a_b
            stats_ref[b, g, 0] = m_n
            stats_ref[b, g, 1] = l_n
            acc_ref[1 - p, b, g, 0:Dh, :] = (
                acc_ref[p, b, g, 0:Dh, :].astype(jnp.float32) * a_a
                + rbuf_ref[s * B + b, g, 0:Dh, :].astype(jnp.float32) * a_b
            ).astype(jnp.bfloat16)
            acc_ref[1 - p, b, g, Dh:Dh + 1, :] = m_n.astype(jnp.bfloat16)
            acc_ref[1 - p, b, g, Dh + 1:Dh + 2, :] = l_n.astype(jnp.bfloat16)

        def project(b):
            slot = N_STAGES % 2
            for h in range(Hq):
                g, hh = divmod(h, GROUP)
                inv_l = 1.0 / stats_ref[b, g, 1, :, hh * Sq:(hh + 1) * Sq]
                obuf_ref[h * Dh:(h + 1) * Dh, :] = (
                    acc_ref[slot, b, g, 0:Dh, hh * Sq:(hh + 1) * Sq].astype(
                        jnp.float32
                    ) * inv_l
                )
            out_ref[b] = lax.dot_general(
                obuf_ref[...], wo_ref[...], (((0,), (0,)), ((), ())),
            )

        if _SKIP_COMM:
            partial(0)
            partial(1)
            project(0)
            project(1)
            return

        partial(0)
        pl.semaphore_wait(barrier_sem, N_STAGES)
        r = [{} for _ in range(N_STAGES)]
        r[0][(0, 0)] = send(0, 0, 0)
        r[0][(0, 1)] = send(0, 0, 1)
        partial(1)
        r[0][(1, 0)] = send(0, 1, 0)
        r[0][(1, 1)] = send(0, 1, 1)
        for s in range(N_STAGES - 1):
            for bg in STREAMS:
                combine(s, *bg, r[s][bg], r[s - 1][bg] if s else None)
                r[s + 1][bg] = send(s + 1, *bg)
        last = N_STAGES - 1
        combine(last, 0, 0, r[last][(0, 0)], r[last - 1][(0, 0)])
        combine(last, 0, 1, r[last][(0, 1)], r[last - 1][(0, 1)])
        project(0)
        combine(last, 1, 0, r[last][(1, 0)], r[last - 1][(1, 0)])
        combine(last, 1, 1, r[last][(1, 1)], r[last - 1][(1, 1)])
        project(1)
        for bg in STREAMS:
            r[last][bg].wait_send()

    return pl.pallas_call(
        body,
        out_shape=jax.ShapeDtypeStruct((B, Sq, D), jnp.float32),
        in_specs=[pl.BlockSpec(memory_space=pltpu.VMEM)] * 5,
        out_specs=pl.BlockSpec(memory_space=pltpu.VMEM),
        scratch_shapes=[
            pltpu.VMEM((2, B, Hkv, PR, GSQ), jnp.bfloat16),
            pltpu.VMEM((B, Hkv, 2, 1, GSQ), jnp.float32),
            pltpu.VMEM((N_STAGES * B, Hkv, PR, GSQ), jnp.bfloat16),
            pltpu.VMEM((B, Hkv, Sq, Dh), jnp.float32),
            pltpu.VMEM((B, Hkv, Sq, Dh), jnp.float32),
            pltpu.VMEM((B, Hkv, GSQ, Dh), jnp.float32),
            pltpu.VMEM((D, Sq), jnp.float32),
            pltpu.SemaphoreType.DMA((N_STAGES, B, Hkv)),
            pltpu.SemaphoreType.DMA((N_STAGES, B, Hkv)),
        ],
        compiler_params=(
            None if _SKIP_COMM else pltpu.CompilerParams(collective_id=0)
        ),
    )(x, Wq, Wo, K_ext, V_ext)


# device time: 23612 ns/iter; 1.0897x vs baseline; 1.0897x over previous
import os

import jax
import jax.numpy as jnp
from jax import lax
from jax.experimental import pallas as pl
from jax.experimental.pallas import tpu as pltpu

_SKIP_COMM = os.environ.get("SKIP_COMM") == "1"
_SKIP_RDMA = os.environ.get("SKIP_RDMA") == "1"

N_DEV = 16
BMASKS = ((8, 3, 4, 1), (3, 8, 1, 4))
BARRIER_MASKS = (8, 3, 4, 1)
N_STAGES = 4

B, Sq, Hq, Hkv, Dh = 2, 128, 8, 2, 64
D = Hq * Dh
GROUP = Hq // Hkv
GSQ = GROUP * Sq
SCALE = 0.125
STREAMS = tuple((b, g) for b in range(B) for g in range(Hkv))


def kernel(x, Wq, Wo, K_ext, V_ext):
    def body(x_ref, wq_ref, wo_ref, k_ref, v_ref, out_ref,
             acc_ref, stats_ref, rbuf_ref, rstats_ref,
             kbuf_ref, vbuf_ref, qg_ref, obuf_ref,
             send_o, recv_o, send_s, recv_s):
        my = lax.axis_index("i")

        if not (_SKIP_COMM or _SKIP_RDMA):
            barrier_sem = pltpu.get_barrier_semaphore()
            for mask in BARRIER_MASKS:
                pl.semaphore_signal(
                    barrier_sem, inc=1,
                    device_id=(my ^ mask,),
                    device_id_type=pl.DeviceIdType.MESH,
                )

        def partial(b):
            for g in range(Hkv):
                kbuf_ref[b, g] = k_ref[b, :, g, :]
                vbuf_ref[b, g] = v_ref[b, :, g, :]
            q_all = lax.dot_general(
                x_ref[b], wq_ref[...], (((1,), (0,)), ((), ())),
            ) * SCALE
            for h in range(Hq):
                g, hh = divmod(h, GROUP)
                qg_ref[b, g, hh * Sq:(hh + 1) * Sq, :] = (
                    q_all[:, h * Dh:(h + 1) * Dh]
                )
            st = lax.dot_general(
                kbuf_ref[b], qg_ref[b], (((2,), (2,)), ((0,), (0,))),
            )
            m = jnp.max(st, axis=1, keepdims=True)
            p = jnp.exp(st - m)
            l = jnp.sum(p, axis=1, keepdims=True)
            ot = lax.dot_general(
                vbuf_ref[b], p, (((1,), (1,)), ((0,), (0,))),
            )
            acc_ref[0, b] = ot.astype(jnp.bfloat16)
            stats_ref[b, :, 0] = m
            stats_ref[b, :, 1] = l

        def send(s, b, g):
            if _SKIP_RDMA:
                return None
            partner = my ^ BMASKS[b][s]
            st_rdma = pltpu.make_async_remote_copy(
                src_ref=stats_ref.at[b, g],
                dst_ref=rstats_ref.at[s * B + b, g],
                send_sem=send_s.at[s, b, g],
                recv_sem=recv_s.at[s, b, g],
                device_id=(partner,),
                device_id_type=pl.DeviceIdType.MESH,
            )
            out_rdma = pltpu.make_async_remote_copy(
                src_ref=acc_ref.at[s % 2, b, g],
                dst_ref=rbuf_ref.at[s * B + b, g],
                send_sem=send_o.at[s, b, g],
                recv_sem=recv_o.at[s, b, g],
                device_id=(partner,),
                device_id_type=pl.DeviceIdType.MESH,
            )
            st_rdma.start()
            out_rdma.start()
            return st_rdma, out_rdma

        def combine(s, b, g, rdmas, prev):
            p = s % 2
            if not _SKIP_RDMA:
                st_rdma, out_rdma = rdmas
                st_rdma.wait()
            m_a = stats_ref[b, g, 0]
            l_a = stats_ref[b, g, 1]
            m_b = rstats_ref[s * B + b, g, 0]
            l_b = rstats_ref[s * B + b, g, 1]
            m_n = jnp.maximum(m_a, m_b)
            a_a = jnp.exp(m_a - m_n)
            a_b = jnp.exp(m_b - m_n)
            stats_ref[b, g, 0] = m_n
            stats_ref[b, g, 1] = l_a * a_a + l_b * a_b
            if not _SKIP_RDMA:
                if prev is not None:
                    prev[1].wait_send()
                out_rdma.wait_recv()
            acc_ref[1 - p, b, g] = (
                acc_ref[p, b, g].astype(jnp.float32) * a_a
                + rbuf_ref[s * B + b, g].astype(jnp.float32) * a_b
            ).astype(jnp.bfloat16)

        def project(b):
            slot = N_STAGES % 2
            for h in range(Hq):
                g, hh = divmod(h, GROUP)
                inv_l = 1.0 / stats_ref[b, g, 1, :, hh * Sq:(hh + 1) * Sq]
                obuf_ref[h * Dh:(h + 1) * Dh, :] = (
                    acc_ref[slot, b, g, :, hh * Sq:(hh + 1) * Sq].astype(
                        jnp.float32
                    ) * inv_l
                )
            out_ref[b] = lax.dot_general(
                obuf_ref[...], wo_ref[...], (((0,), (0,)), ((), ())),
            )

        if _SKIP_COMM:
            partial(0)
            partial(1)
            project(0)
            project(1)
            return

        partial(0)
        if not _SKIP_RDMA:
            pl.semaphore_wait(barrier_sem, N_STAGES)
        r = [{} for _ in range(N_STAGES)]
        r[0][(0, 0)] = send(0, 0, 0)
        r[0][(0, 1)] = send(0, 0, 1)
        partial(1)
        r[0][(1, 0)] = send(0, 1, 0)
        r[0][(1, 1)] = send(0, 1, 1)
        for s in range(N_STAGES - 1):
            for bg in STREAMS:
                combine(s, *bg, r[s][bg], r[s - 1][bg] if s else None)
                r[s + 1][bg] = send(s + 1, *bg)
        last = N_STAGES - 1
        combine(last, 0, 0, r[last][(0, 0)], r[last - 1][(0, 0)])
        combine(last, 0, 1, r[last][(0, 1)], r[last - 1][(0, 1)])
        project(0)
        combine(last, 1, 0, r[last][(1, 0)], r[last - 1][(1, 0)])
        combine(last, 1, 1, r[last][(1, 1)], r[last - 1][(1, 1)])
        project(1)
        if not _SKIP_RDMA:
            for bg in STREAMS:
                r[last][bg][1].wait_send()

    return pl.pallas_call(
        body,
        out_shape=jax.ShapeDtypeStruct((B, Sq, D), jnp.float32),
        in_specs=[pl.BlockSpec(memory_space=pltpu.VMEM)] * 5,
        out_specs=pl.BlockSpec(memory_space=pltpu.VMEM),
        scratch_shapes=[
            pltpu.VMEM((2, B, Hkv, Dh, GSQ), jnp.bfloat16),
            pltpu.VMEM((B, Hkv, 2, 1, GSQ), jnp.float32),
            pltpu.VMEM((N_STAGES * B, Hkv, Dh, GSQ), jnp.bfloat16),
            pltpu.VMEM((N_STAGES * B, Hkv, 2, 1, GSQ), jnp.float32),
            pltpu.VMEM((B, Hkv, Sq, Dh), jnp.float32),
            pltpu.VMEM((B, Hkv, Sq, Dh), jnp.float32),
            pltpu.VMEM((B, Hkv, GSQ, Dh), jnp.float32),
            pltpu.VMEM((D, Sq), jnp.float32),
            pltpu.SemaphoreType.DMA((N_STAGES, B, Hkv)),
            pltpu.SemaphoreType.DMA((N_STAGES, B, Hkv)),
            pltpu.SemaphoreType.DMA((N_STAGES, B, Hkv)),
            pltpu.SemaphoreType.DMA((N_STAGES, B, Hkv)),
        ],
        compiler_params=(
            None if (_SKIP_COMM or _SKIP_RDMA)
            else pltpu.CompilerParams(collective_id=0)
        ),
    )(x, Wq, Wo, K_ext, V_ext)


# device time: 22956 ns/iter; 1.1209x vs baseline; 1.0286x over previous
import os

import jax
import jax.numpy as jnp
from jax import lax
from jax.experimental import pallas as pl
from jax.experimental.pallas import tpu as pltpu

_SKIP_COMM = os.environ.get("SKIP_COMM") == "1"
_SKIP_RDMA = os.environ.get("SKIP_RDMA") == "1"

N_DEV = 16
SMASKS = {
    (0, 0): (8, 3, 4, 1),
    (0, 1): (4, 1, 8, 3),
    (1, 0): (3, 8, 1, 4),
    (1, 1): (1, 4, 3, 8),
}
BARRIER_MASKS = (8, 3, 4, 1)
N_STAGES = 4

B, Sq, Hq, Hkv, Dh = 2, 128, 8, 2, 64
D = Hq * Dh
GROUP = Hq // Hkv
GSQ = GROUP * Sq
SCALE = 0.125
STREAMS = tuple((b, g) for b in range(B) for g in range(Hkv))


def kernel(x, Wq, Wo, K_ext, V_ext):
    def body(x_ref, wq_ref, wo_ref, k_ref, v_ref, out_ref,
             acc_ref, stats_ref, rbuf_ref, rstats_ref,
             kbuf_ref, vbuf_ref, qg_ref, obuf_ref,
             send_o, recv_o, send_s, recv_s):
        my = lax.axis_index("i")

        if not (_SKIP_COMM or _SKIP_RDMA):
            barrier_sem = pltpu.get_barrier_semaphore()
            for mask in BARRIER_MASKS:
                pl.semaphore_signal(
                    barrier_sem, inc=1,
                    device_id=(my ^ mask,),
                    device_id_type=pl.DeviceIdType.MESH,
                )

        def partial(b):
            for g in range(Hkv):
                kbuf_ref[b, g] = k_ref[b, :, g, :]
                vbuf_ref[b, g] = v_ref[b, :, g, :]
            q_all = lax.dot_general(
                x_ref[b], wq_ref[...], (((1,), (0,)), ((), ())),
            ) * SCALE
            for h in range(Hq):
                g, hh = divmod(h, GROUP)
                qg_ref[b, g, hh * Sq:(hh + 1) * Sq, :] = (
                    q_all[:, h * Dh:(h + 1) * Dh]
                )
            st = lax.dot_general(
                kbuf_ref[b], qg_ref[b], (((2,), (2,)), ((0,), (0,))),
            )
            m = jnp.max(st, axis=1, keepdims=True)
            p = jnp.exp(st - m)
            l = jnp.sum(p, axis=1, keepdims=True)
            ot = lax.dot_general(
                vbuf_ref[b], p, (((1,), (1,)), ((0,), (0,))),
            )
            acc_ref[0, b] = ot.astype(jnp.bfloat16)
            stats_ref[b, :, 0] = m
            stats_ref[b, :, 1] = l

        def send(s, b, g):
            if _SKIP_RDMA:
                return None
            partner = my ^ SMASKS[(b, g)][s]
            st_rdma = pltpu.make_async_remote_copy(
                src_ref=stats_ref.at[b, g],
                dst_ref=rstats_ref.at[s * B + b, g],
                send_sem=send_s.at[s, b, g],
                recv_sem=recv_s.at[s, b, g],
                device_id=(partner,),
                device_id_type=pl.DeviceIdType.MESH,
            )
            out_rdma = pltpu.make_async_remote_copy(
                src_ref=acc_ref.at[s % 2, b, g],
                dst_ref=rbuf_ref.at[s * B + b, g],
                send_sem=send_o.at[s, b, g],
                recv_sem=recv_o.at[s, b, g],
                device_id=(partner,),
                device_id_type=pl.DeviceIdType.MESH,
            )
            st_rdma.start()
            out_rdma.start()
            return st_rdma, out_rdma

        def combine(s, b, g, rdmas, prev):
            p = s % 2
            if not _SKIP_RDMA:
                st_rdma, out_rdma = rdmas
                st_rdma.wait()
            m_a = stats_ref[b, g, 0]
            l_a = stats_ref[b, g, 1]
            m_b = rstats_ref[s * B + b, g, 0]
            l_b = rstats_ref[s * B + b, g, 1]
            m_n = jnp.maximum(m_a, m_b)
            a_a = jnp.exp(m_a - m_n)
            a_b = jnp.exp(m_b - m_n)
            stats_ref[b, g, 0] = m_n
            stats_ref[b, g, 1] = l_a * a_a + l_b * a_b
            if not _SKIP_RDMA:
                if prev is not None:
                    prev[1].wait_send()
                out_rdma.wait_recv()
            acc_ref[1 - p, b, g] = (
                acc_ref[p, b, g].astype(jnp.float32) * a_a
                + rbuf_ref[s * B + b, g].astype(jnp.float32) * a_b
            ).astype(jnp.bfloat16)

        def project(b):
            slot = N_STAGES % 2
            for h in range(Hq):
                g, hh = divmod(h, GROUP)
                inv_l = 1.0 / stats_ref[b, g, 1, :, hh * Sq:(hh + 1) * Sq]
                obuf_ref[h * Dh:(h + 1) * Dh, :] = (
                    acc_ref[slot, b, g, :, hh * Sq:(hh + 1) * Sq].astype(
                        jnp.float32
                    ) * inv_l
                )
            out_ref[b] = lax.dot_general(
                obuf_ref[...], wo_ref[...], (((0,), (0,)), ((), ())),
            )

        if _SKIP_COMM:
            partial(0)
            partial(1)
            project(0)
            project(1)
            return

        partial(0)
        if not _SKIP_RDMA:
            pl.semaphore_wait(barrier_sem, N_STAGES)
        r = [{} for _ in range(N_STAGES)]
        r[0][(0, 0)] = send(0, 0, 0)
        r[0][(0, 1)] = send(0, 0, 1)
        partial(1)
        r[0][(1, 0)] = send(0, 1, 0)
        r[0][(1, 1)] = send(0, 1, 1)
        for s in range(N_STAGES - 1):
            for bg in STREAMS:
                combine(s, *bg, r[s][bg], r[s - 1][bg] if s else None)
                r[s + 1][bg] = send(s + 1, *bg)
        last = N_STAGES - 1
        combine(last, 0, 0, r[last][(0, 0)], r[last - 1][(0, 0)])
        combine(last, 0, 1, r[last][(0, 1)], r[last - 1][(0, 1)])
        project(0)
        combine(last, 1, 0, r[last][(1, 0)], r[last - 1][(1, 0)])
        combine(last, 1, 1, r[last][(1, 1)], r[last - 1][(1, 1)])
        project(1)
        if not _SKIP_RDMA:
            for bg in STREAMS:
                r[last][bg][1].wait_send()

    return pl.pallas_call(
        body,
        out_shape=jax.ShapeDtypeStruct((B, Sq, D), jnp.float32),
        in_specs=[pl.BlockSpec(memory_space=pltpu.VMEM)] * 5,
        out_specs=pl.BlockSpec(memory_space=pltpu.VMEM),
        scratch_shapes=[
            pltpu.VMEM((2, B, Hkv, Dh, GSQ), jnp.bfloat16),
            pltpu.VMEM((B, Hkv, 2, 1, GSQ), jnp.float32),
            pltpu.VMEM((N_STAGES * B, Hkv, Dh, GSQ), jnp.bfloat16),
            pltpu.VMEM((N_STAGES * B, Hkv, 2, 1, GSQ), jnp.float32),
            pltpu.VMEM((B, Hkv, Sq, Dh), jnp.float32),
            pltpu.VMEM((B, Hkv, Sq, Dh), jnp.float32),
            pltpu.VMEM((B, Hkv, GSQ, Dh), jnp.float32),
            pltpu.VMEM((D, Sq), jnp.float32),
            pltpu.SemaphoreType.DMA((N_STAGES, B, Hkv)),
            pltpu.SemaphoreType.DMA((N_STAGES, B, Hkv)),
            pltpu.SemaphoreType.DMA((N_STAGES, B, Hkv)),
            pltpu.SemaphoreType.DMA((N_STAGES, B, Hkv)),
        ],
        compiler_params=(
            None if (_SKIP_COMM or _SKIP_RDMA)
            else pltpu.CompilerParams(collective_id=0)
        ),
    )(x, Wq, Wo, K_ext, V_ext)
